# Initial kernel scaffold; baseline (speedup 1.0000x reference)
import jax
import jax.numpy as jnp
from jax.experimental import pallas as pl


def kernel(z, W):
    raise NotImplementedError("write your pallas kernel here")


# speedup vs baseline: 1.0497x; 1.0497x over previous
import functools

import jax
import jax.numpy as jnp
from jax import lax
from jax.experimental import pallas as pl
from jax.experimental.pallas import tpu as pltpu
from jax.experimental.pallas import tpu_sc as plsc

_N_E = 8192
_E = 256
_T = 256
_C = 512
_NT = _N_E // _T
_NC = _N_E // _C
_TOK = 8 * 32 * 32


def _norm_body(w_ref, emb_ref, e_ref):
    w = w_ref[...]
    n = jnp.sqrt(jnp.sum(w * w, axis=1, keepdims=True))
    emb = w / jnp.maximum(n, 1e-12)
    emb_ref[...] = emb
    e_ref[0, :] = jnp.sum(emb * emb, axis=1)


def _dist_body(z_ref, emb_ref, e_ref, idx_ref, loss_ref, d_ref):
    i = pl.program_id(0)
    zt = z_ref[...]
    zn = zt / jnp.maximum(jnp.sqrt(jnp.sum(zt * zt, axis=1, keepdims=True)), 1e-12)
    zn2 = jnp.sum(zn * zn, axis=1, keepdims=True)

    def pass1(j, dmin):
        wj = emb_ref[pl.ds(j * _C, _C), :]
        s = lax.dot_general(zn, wj, (((1,), (1,)), ((), ())),
                            preferred_element_type=jnp.float32)
        d = (zn2 + e_ref[0, pl.ds(j * _C, _C)][None, :]) - 2.0 * s
        d_ref[:, pl.ds(j * _C, _C)] = d
        return jnp.minimum(dmin, jnp.min(d, axis=1))

    dmin = lax.fori_loop(0, _NC, pass1,
                         jnp.full((_T,), jnp.inf, jnp.float32))

    def pass2(j, z_acc):
        d = d_ref[:, pl.ds(j * _C, _C)]
        u = jnp.exp((dmin[:, None] - d) * 0.5)
        d_ref[:, pl.ds(j * _C, _C)] = u
        return z_acc + jnp.sum(u, axis=1)

    zsum = lax.fori_loop(0, _NC, pass2, jnp.zeros((_T,), jnp.float32))
    pmax = 1.0 / zsum

    def pass3(j, best):
        u = d_ref[:, pl.ds(j * _C, _C)]
        p = u / zsum[:, None]
        cols = lax.broadcasted_iota(jnp.int32, (_T, _C), 1) + j * _C
        cand = jnp.where(p == pmax[:, None], cols, _N_E)
        return jnp.minimum(best, jnp.min(cand, axis=1))

    idx = lax.fori_loop(0, _NC, pass3,
                        jnp.full((_T,), _N_E, jnp.int32))
    idx_ref[0, 0, :] = idx

    @pl.when(i == 0)
    def _init():
        loss_ref[0, 0] = 0.0

    loss_ref[0, 0] += jnp.sum(dmin)

    @pl.when(i == _NT - 1)
    def _fin():
        loss_ref[0, 0] = loss_ref[0, 0] * (2.25 / float(_TOK * _E))


_norm_call = pl.pallas_call(
    _norm_body,
    grid=(16,),
    in_specs=[pl.BlockSpec((_N_E // 16, _E), lambda j: (j, 0))],
    out_specs=[pl.BlockSpec((_N_E // 16, _E), lambda j: (j, 0)),
               pl.BlockSpec((1, _N_E // 16), lambda j: (0, j))],
    out_shape=[jax.ShapeDtypeStruct((_N_E, _E), jnp.float32),
               jax.ShapeDtypeStruct((1, _N_E), jnp.float32)],
)

_dist_call = pl.pallas_call(
    _dist_body,
    grid=(_NT,),
    in_specs=[pl.BlockSpec((_T, _E), lambda i: (i, 0)),
              pl.BlockSpec((_N_E, _E), lambda i: (0, 0)),
              pl.BlockSpec((1, _N_E), lambda i: (0, 0))],
    out_specs=[pl.BlockSpec((1, 1, _T), lambda i: (i, 0, 0)),
               pl.BlockSpec((1, 1), lambda i: (0, 0),
                            memory_space=pltpu.SMEM)],
    out_shape=[jax.ShapeDtypeStruct((_NT, 1, _T), jnp.int32),
               jax.ShapeDtypeStruct((1, 1), jnp.float32)],
    scratch_shapes=[pltpu.VMEM((_T, _N_E), jnp.float32)],
    compiler_params=pltpu.CompilerParams(
        dimension_semantics=("arbitrary",)),
)


def _make_gather():
    info = plsc.get_sparse_core_info()
    nw = info.num_cores * info.num_subcores
    bpw = _N_E // nw
    mesh = plsc.VectorSubcoreMesh(core_axis_name="c", subcore_axis_name="s")

    @functools.partial(
        pl.kernel, mesh=mesh,
        out_type=jax.ShapeDtypeStruct((_N_E, _E), jnp.float32),
        scratch_types=[
            pltpu.VMEM((bpw,), jnp.int32),
            pltpu.VMEM((bpw, _E), jnp.float32),
            pltpu.SemaphoreType.DMA,
        ],
    )
    def _gather(table_hbm, idx_hbm, out_hbm, idx_v, rows_v, sem):
        wid = lax.axis_index("s") * info.num_cores + lax.axis_index("c")
        base = wid * bpw
        pltpu.sync_copy(idx_hbm.at[pl.ds(base, bpw)], idx_v)
        pltpu.async_copy(table_hbm.at[idx_v], rows_v, sem).wait()
        pltpu.sync_copy(rows_v, out_hbm.at[pl.ds(base, bpw)])

    return _gather


def kernel(z, W):
    z_flat = jnp.transpose(z, (0, 2, 3, 1)).reshape(_TOK, _E)
    emb, e = _norm_call(W)
    idx3, loss = _dist_call(z_flat, emb, e)
    idx = idx3.reshape(_TOK)
    zq_rows = _make_gather()(W, idx)
    z_q = zq_rows.reshape(8, 32, 32, _E).transpose(0, 3, 1, 2)
    return z_q, idx.reshape(-1, 1), loss.reshape(())


# speedup vs baseline: 1.3940x; 1.3280x over previous
import functools

import jax
import jax.numpy as jnp
from jax import lax
from jax.experimental import pallas as pl
from jax.experimental.pallas import tpu as pltpu
from jax.experimental.pallas import tpu_sc as plsc

_N_E = 8192
_E = 256
_T = 256
_C = 512
_NT = _N_E // _T
_NC = _N_E // _C
_TOK = 8 * 32 * 32


def _norm_body(w_ref, emb_ref, e_ref):
    w = w_ref[...]
    n = jnp.sqrt(jnp.sum(w * w, axis=1, keepdims=True))
    emb = w / jnp.maximum(n, 1e-12)
    emb_ref[...] = emb
    e_ref[0, :] = jnp.sum(emb * emb, axis=1)


def _dist_body(z_ref, emb_ref, e_ref, idx_ref, loss_ref, d_ref):
    i = pl.program_id(0)
    zt = z_ref[...]
    zn = zt / jnp.maximum(jnp.sqrt(jnp.sum(zt * zt, axis=1, keepdims=True)), 1e-12)
    zn2 = jnp.sum(zn * zn, axis=1, keepdims=True)
    cols0 = lax.broadcasted_iota(jnp.int32, (_T, _C), 1)

    def pass1(j, carry):
        m1, m2, idx = carry
        wj = emb_ref[pl.ds(j * _C, _C), :]
        s = lax.dot_general(zn, wj, (((1,), (1,)), ((), ())),
                            preferred_element_type=jnp.float32)
        q = e_ref[0, pl.ds(j * _C, _C)][None, :] * 0.5 - s
        qmin_c = jnp.min(q, axis=1)
        t = q == qmin_c[:, None]
        idx_c = jnp.min(jnp.where(t, cols0 + j * _C, _N_E), axis=1)
        q2_c = jnp.min(jnp.where(t, jnp.inf, q), axis=1)
        new_m1 = jnp.minimum(m1, qmin_c)
        new_m2 = jnp.minimum(jnp.minimum(m2, q2_c), jnp.maximum(m1, qmin_c))
        new_idx = jnp.where(qmin_c < m1, idx_c, idx)
        return new_m1, new_m2, new_idx

    m1, m2, idx_fast = lax.fori_loop(
        0, _NC, pass1,
        (jnp.full((_T,), jnp.inf, jnp.float32),
         jnp.full((_T,), jnp.inf, jnp.float32),
         jnp.zeros((_T,), jnp.int32)))

    dmin_fast = zn2[:, 0] + 2.0 * m1
    has_tie = jnp.any(m2 < m1 + 5e-7)

    def _slow():
        def sp1(j, dmin):
            wj = emb_ref[pl.ds(j * _C, _C), :]
            s = lax.dot_general(zn, wj, (((1,), (1,)), ((), ())),
                                preferred_element_type=jnp.float32)
            d = (zn2 + e_ref[0, pl.ds(j * _C, _C)][None, :]) - 2.0 * s
            d_ref[:, pl.ds(j * _C, _C)] = d
            return jnp.minimum(dmin, jnp.min(d, axis=1))

        dmin = lax.fori_loop(0, _NC, sp1,
                             jnp.full((_T,), jnp.inf, jnp.float32))

        def sp2(j, z_acc):
            d = d_ref[:, pl.ds(j * _C, _C)]
            u = jnp.exp((dmin[:, None] - d) * 0.5)
            d_ref[:, pl.ds(j * _C, _C)] = u
            return z_acc + jnp.sum(u, axis=1)

        zsum = lax.fori_loop(0, _NC, sp2, jnp.zeros((_T,), jnp.float32))
        pmax = 1.0 / zsum

        def sp3(j, best):
            u = d_ref[:, pl.ds(j * _C, _C)]
            p = u / zsum[:, None]
            cand = jnp.where(p == pmax[:, None], cols0 + j * _C, _N_E)
            return jnp.minimum(best, jnp.min(cand, axis=1))

        return lax.fori_loop(0, _NC, sp3,
                             jnp.full((_T,), _N_E, jnp.int32))

    idx = lax.cond(has_tie, _slow, lambda: idx_fast)
    idx_ref[0, 0, :] = idx

    @pl.when(i == 0)
    def _init():
        loss_ref[0, 0] = 0.0

    loss_ref[0, 0] += jnp.sum(dmin_fast)

    @pl.when(i == _NT - 1)
    def _fin():
        loss_ref[0, 0] = loss_ref[0, 0] * (2.25 / float(_TOK * _E))


_norm_call = pl.pallas_call(
    _norm_body,
    grid=(16,),
    in_specs=[pl.BlockSpec((_N_E // 16, _E), lambda j: (j, 0))],
    out_specs=[pl.BlockSpec((_N_E // 16, _E), lambda j: (j, 0)),
               pl.BlockSpec((1, _N_E // 16), lambda j: (0, j))],
    out_shape=[jax.ShapeDtypeStruct((_N_E, _E), jnp.float32),
               jax.ShapeDtypeStruct((1, _N_E), jnp.float32)],
)

_dist_call = pl.pallas_call(
    _dist_body,
    grid=(_NT,),
    in_specs=[pl.BlockSpec((_T, _E), lambda i: (i, 0)),
              pl.BlockSpec((_N_E, _E), lambda i: (0, 0)),
              pl.BlockSpec((1, _N_E), lambda i: (0, 0))],
    out_specs=[pl.BlockSpec((1, 1, _T), lambda i: (i, 0, 0)),
               pl.BlockSpec((1, 1), lambda i: (0, 0),
                            memory_space=pltpu.SMEM)],
    out_shape=[jax.ShapeDtypeStruct((_NT, 1, _T), jnp.int32),
               jax.ShapeDtypeStruct((1, 1), jnp.float32)],
    scratch_shapes=[pltpu.VMEM((_T, _N_E), jnp.float32)],
    compiler_params=pltpu.CompilerParams(
        dimension_semantics=("arbitrary",)),
)


def _make_gather():
    info = plsc.get_sparse_core_info()
    nw = info.num_cores * info.num_subcores
    bpw = _N_E // nw
    mesh = plsc.VectorSubcoreMesh(core_axis_name="c", subcore_axis_name="s")

    @functools.partial(
        pl.kernel, mesh=mesh,
        out_type=jax.ShapeDtypeStruct((_N_E, _E), jnp.float32),
        scratch_types=[
            pltpu.VMEM((bpw,), jnp.int32),
            pltpu.VMEM((bpw, _E), jnp.float32),
            pltpu.SemaphoreType.DMA,
        ],
    )
    def _gather(table_hbm, idx_hbm, out_hbm, idx_v, rows_v, sem):
        wid = lax.axis_index("s") * info.num_cores + lax.axis_index("c")
        base = wid * bpw
        pltpu.sync_copy(idx_hbm.at[pl.ds(base, bpw)], idx_v)
        pltpu.async_copy(table_hbm.at[idx_v], rows_v, sem).wait()
        pltpu.sync_copy(rows_v, out_hbm.at[pl.ds(base, bpw)])

    return _gather


def kernel(z, W):
    z_flat = jnp.transpose(z, (0, 2, 3, 1)).reshape(_TOK, _E)
    emb, e = _norm_call(W)
    idx3, loss = _dist_call(z_flat, emb, e)
    idx = idx3.reshape(_TOK)
    zq_rows = _make_gather()(W, idx)
    z_q = zq_rows.reshape(8, 32, 32, _E).transpose(0, 3, 1, 2)
    return z_q, idx.reshape(-1, 1), loss.reshape(())
